# baseline (device time: 30231 ns/iter reference)
import jax
import jax.numpy as jnp
from jax import lax
from jax.experimental import pallas as pl
from jax.experimental.pallas import tpu as pltpu

N_DEV = 32
B, SQ, D_MODEL = 2, 128, 512
HQ_LOCAL, DH = 4, 64
HD_LOCAL = HQ_LOCAL * DH
ROWS = B * SQ
CHUNK = ROWS // N_DEV


def kernel(x, Wq, K_ext, V_ext, Wo):
    me_out = lax.axis_index("i")
    Wq_s = lax.dynamic_slice(Wq, (0, me_out * HD_LOCAL), (D_MODEL, HD_LOCAL))
    Wo_s = lax.dynamic_slice(Wo, (me_out * HD_LOCAL, 0), (HD_LOCAL, D_MODEL))
    x2 = x.reshape(ROWS, D_MODEL)

    def body(x_ref, wq_ref, k_ref, v_ref, wo_ref, out_ref,
             part_ref, rs_ref, ssem1, rsem1, ssem2, rsem2):
        me = lax.axis_index("i")

        barrier = pltpu.get_barrier_semaphore()
        for t in range(1, N_DEV):
            j = lax.rem(me + t, N_DEV)
            pl.semaphore_signal(
                barrier, inc=1,
                device_id=(j,), device_id_type=pl.DeviceIdType.MESH,
            )

        Q = jnp.dot(x_ref[...], wq_ref[...],
                    preferred_element_type=jnp.float32)
        brows = []
        for b in range(B):
            heads = []
            for h in range(HQ_LOCAL):
                q = Q[b * SQ:(b + 1) * SQ, h * DH:(h + 1) * DH]
                k = k_ref[b, :, h, :]
                v = v_ref[b, :, h, :]
                s = lax.dot_general(
                    q, k, (((1,), (1,)), ((), ())),
                    preferred_element_type=jnp.float32) * 0.125
                m = jnp.max(s, axis=1, keepdims=True)
                w = jnp.exp(s - m)
                w = w / jnp.sum(w, axis=1, keepdims=True)
                heads.append(jnp.dot(w, v,
                                     preferred_element_type=jnp.float32))
            brows.append(jnp.concatenate(heads, axis=1))
        ctx = jnp.concatenate(brows, axis=0)
        part_ref[...] = jnp.dot(ctx, wo_ref[...],
                                preferred_element_type=jnp.float32)

        rs_ref[pl.ds(me, 1)] = part_ref[pl.ds(me * CHUNK, CHUNK)].reshape(
            1, CHUNK, D_MODEL)

        pl.semaphore_wait(barrier, N_DEV - 1)

        sends1 = []
        for t in range(1, N_DEV):
            j = lax.rem(me + t, N_DEV)
            r = pltpu.make_async_remote_copy(
                src_ref=part_ref.at[pl.ds(j * CHUNK, CHUNK)],
                dst_ref=rs_ref.at[me],
                send_sem=ssem1.at[t - 1],
                recv_sem=rsem1.at[me],
                device_id=(j,),
                device_id_type=pl.DeviceIdType.MESH,
            )
            r.start()
            sends1.append(r)
        for t in range(1, N_DEV):
            j = lax.rem(me + t, N_DEV)
            rr = pltpu.make_async_remote_copy(
                src_ref=part_ref.at[pl.ds(0, CHUNK)],
                dst_ref=rs_ref.at[j],
                send_sem=ssem1.at[t - 1],
                recv_sem=rsem1.at[j],
                device_id=(j,),
                device_id_type=pl.DeviceIdType.MESH,
            )
            rr.wait_recv()

        out_ref[pl.ds(me * CHUNK, CHUNK)] = jnp.sum(rs_ref[...], axis=0)

        sends2 = []
        for t in range(1, N_DEV):
            j = lax.rem(me + t, N_DEV)
            r = pltpu.make_async_remote_copy(
                src_ref=out_ref.at[pl.ds(me * CHUNK, CHUNK)],
                dst_ref=out_ref.at[pl.ds(me * CHUNK, CHUNK)],
                send_sem=ssem2.at[t - 1],
                recv_sem=rsem2.at[me],
                device_id=(j,),
                device_id_type=pl.DeviceIdType.MESH,
            )
            r.start()
            sends2.append(r)
        for t in range(1, N_DEV):
            j = lax.rem(me + t, N_DEV)
            rr = pltpu.make_async_remote_copy(
                src_ref=out_ref.at[pl.ds(me * CHUNK, CHUNK)],
                dst_ref=out_ref.at[pl.ds(j * CHUNK, CHUNK)],
                send_sem=ssem2.at[t - 1],
                recv_sem=rsem2.at[j],
                device_id=(j,),
                device_id_type=pl.DeviceIdType.MESH,
            )
            rr.wait_recv()

        for r in sends1:
            r.wait_send()
        for r in sends2:
            r.wait_send()

    out2 = pl.pallas_call(
        body,
        out_shape=jax.ShapeDtypeStruct((ROWS, D_MODEL), jnp.float32),
        in_specs=[pl.BlockSpec(memory_space=pltpu.VMEM)] * 5,
        out_specs=pl.BlockSpec(memory_space=pltpu.VMEM),
        scratch_shapes=[
            pltpu.VMEM((ROWS, D_MODEL), jnp.float32),
            pltpu.VMEM((N_DEV, CHUNK, D_MODEL), jnp.float32),
            pltpu.SemaphoreType.DMA((N_DEV - 1,)),
            pltpu.SemaphoreType.DMA((N_DEV,)),
            pltpu.SemaphoreType.DMA((N_DEV - 1,)),
            pltpu.SemaphoreType.DMA((N_DEV,)),
        ],
        compiler_params=pltpu.CompilerParams(collective_id=0),
    )(x2, Wq_s, K_ext, V_ext, Wo_s)
    return out2.reshape(B, SQ, D_MODEL)


# device time: 26641 ns/iter; 1.1348x vs baseline; 1.1348x over previous
import jax
import jax.numpy as jnp
from jax import lax
from jax.experimental import pallas as pl
from jax.experimental.pallas import tpu as pltpu

N_DEV = 32
B, SQ, D_MODEL = 2, 128, 512
HQ_LOCAL, DH = 4, 64
HD_LOCAL = HQ_LOCAL * DH
ROWS = B * SQ
CHUNK = ROWS // N_DEV


def kernel(x, Wq, K_ext, V_ext, Wo):
    me_out = lax.axis_index("i")
    Wq_s = lax.dynamic_slice(Wq, (0, me_out * HD_LOCAL), (D_MODEL, HD_LOCAL))
    Wo_s = lax.dynamic_slice(Wo, (me_out * HD_LOCAL, 0), (HD_LOCAL, D_MODEL))
    x2 = x.reshape(ROWS, D_MODEL)

    def body(x_ref, wq_ref, k_ref, v_ref, wo_ref, out_ref,
             pbf_ref, rs_ref, red_ref, ag_ref, ssem1, rsem1, ssem2, rsem2):
        me = lax.axis_index("i")

        barrier = pltpu.get_barrier_semaphore()
        for t in range(1, N_DEV):
            j = lax.rem(me + t, N_DEV)
            pl.semaphore_signal(
                barrier, inc=1,
                device_id=(j,), device_id_type=pl.DeviceIdType.MESH,
            )

        Q = jnp.dot(x_ref[...], wq_ref[...],
                    preferred_element_type=jnp.float32)
        brows = []
        for b in range(B):
            heads = []
            for h in range(HQ_LOCAL):
                q = Q[b * SQ:(b + 1) * SQ, h * DH:(h + 1) * DH]
                k = k_ref[b, :, h, :]
                v = v_ref[b, :, h, :]
                s = lax.dot_general(
                    q, k, (((1,), (1,)), ((), ())),
                    preferred_element_type=jnp.float32) * 0.125
                m = jnp.max(s, axis=1, keepdims=True)
                w = jnp.exp(s - m)
                w = w / jnp.sum(w, axis=1, keepdims=True)
                heads.append(jnp.dot(w, v,
                                     preferred_element_type=jnp.float32))
            brows.append(jnp.concatenate(heads, axis=1))
        ctx = jnp.concatenate(brows, axis=0)
        part = jnp.dot(ctx, wo_ref[...],
                       preferred_element_type=jnp.float32)
        pbf_ref[...] = part.astype(jnp.bfloat16).reshape(
            N_DEV, CHUNK, D_MODEL)

        rs_ref[pl.ds(me, 1)] = pbf_ref[pl.ds(me, 1)]

        pl.semaphore_wait(barrier, N_DEV - 1)

        sends1 = []
        for t in range(1, N_DEV):
            j = lax.rem(me + t, N_DEV)
            r = pltpu.make_async_remote_copy(
                src_ref=pbf_ref.at[j],
                dst_ref=rs_ref.at[me],
                send_sem=ssem1.at[t - 1],
                recv_sem=rsem1.at[me],
                device_id=(j,),
                device_id_type=pl.DeviceIdType.MESH,
            )
            r.start()
            sends1.append(r)
        for t in range(1, N_DEV):
            j = lax.rem(me + t, N_DEV)
            rr = pltpu.make_async_remote_copy(
                src_ref=pbf_ref.at[0],
                dst_ref=rs_ref.at[j],
                send_sem=ssem1.at[t - 1],
                recv_sem=rsem1.at[j],
                device_id=(j,),
                device_id_type=pl.DeviceIdType.MESH,
            )
            rr.wait_recv()

        red = jnp.sum(rs_ref[...].astype(jnp.float32), axis=0)
        red_ref[...] = red.astype(jnp.bfloat16)

        sends2 = []
        for t in range(1, N_DEV):
            j = lax.rem(me + t, N_DEV)
            r = pltpu.make_async_remote_copy(
                src_ref=red_ref,
                dst_ref=ag_ref.at[me],
                send_sem=ssem2.at[t - 1],
                recv_sem=rsem2.at[me],
                device_id=(j,),
                device_id_type=pl.DeviceIdType.MESH,
            )
            r.start()
            sends2.append(r)
        for t in range(1, N_DEV):
            j = lax.rem(me + t, N_DEV)
            rr = pltpu.make_async_remote_copy(
                src_ref=red_ref,
                dst_ref=ag_ref.at[j],
                send_sem=ssem2.at[t - 1],
                recv_sem=rsem2.at[j],
                device_id=(j,),
                device_id_type=pl.DeviceIdType.MESH,
            )
            rr.wait_recv()

        out_ref[...] = ag_ref[...].astype(jnp.float32).reshape(
            ROWS, D_MODEL)
        out_ref[pl.ds(me * CHUNK, CHUNK)] = red

        for r in sends1:
            r.wait_send()
        for r in sends2:
            r.wait_send()

    out2 = pl.pallas_call(
        body,
        out_shape=jax.ShapeDtypeStruct((ROWS, D_MODEL), jnp.float32),
        in_specs=[pl.BlockSpec(memory_space=pltpu.VMEM)] * 5,
        out_specs=pl.BlockSpec(memory_space=pltpu.VMEM),
        scratch_shapes=[
            pltpu.VMEM((N_DEV, CHUNK, D_MODEL), jnp.bfloat16),
            pltpu.VMEM((N_DEV, CHUNK, D_MODEL), jnp.bfloat16),
            pltpu.VMEM((CHUNK, D_MODEL), jnp.bfloat16),
            pltpu.VMEM((N_DEV, CHUNK, D_MODEL), jnp.bfloat16),
            pltpu.SemaphoreType.DMA((N_DEV - 1,)),
            pltpu.SemaphoreType.DMA((N_DEV,)),
            pltpu.SemaphoreType.DMA((N_DEV - 1,)),
            pltpu.SemaphoreType.DMA((N_DEV,)),
        ],
        compiler_params=pltpu.CompilerParams(collective_id=0),
    )(x2, Wq_s, K_ext, V_ext, Wo_s)
    return out2.reshape(B, SQ, D_MODEL)


# device time: 26621 ns/iter; 1.1356x vs baseline; 1.0008x over previous
import jax
import jax.numpy as jnp
from jax import lax
from jax.experimental import pallas as pl
from jax.experimental.pallas import tpu as pltpu

N_DEV = 32
B, SQ, D_MODEL = 2, 128, 512
HQ_LOCAL, DH = 4, 64
HD_LOCAL = HQ_LOCAL * DH
ROWS = B * SQ
CHUNK = ROWS // N_DEV


def kernel(x, Wq, K_ext, V_ext, Wo):
    me_out = lax.axis_index("i")
    Wq_s = lax.dynamic_slice(Wq, (0, me_out * HD_LOCAL), (D_MODEL, HD_LOCAL))
    Wo_s = lax.dynamic_slice(Wo, (me_out * HD_LOCAL, 0), (HD_LOCAL, D_MODEL))
    Wq_b = Wq_s.astype(jnp.bfloat16)
    Wo_b = Wo_s.astype(jnp.bfloat16)
    K_b = K_ext.astype(jnp.bfloat16)
    V_b = V_ext.astype(jnp.bfloat16)
    x2 = x.reshape(ROWS, D_MODEL).astype(jnp.bfloat16)

    def body(x_ref, wq_ref, k_ref, v_ref, wo_ref, out_ref,
             pbf_ref, rs_ref, red_ref, ag_ref, ssem1, rsem1, ssem2, rsem2):
        me = lax.axis_index("i")

        barrier = pltpu.get_barrier_semaphore()
        for t in range(1, N_DEV):
            j = lax.rem(me + t, N_DEV)
            pl.semaphore_signal(
                barrier, inc=1,
                device_id=(j,), device_id_type=pl.DeviceIdType.MESH,
            )

        Q = jnp.dot(x_ref[...], wq_ref[...],
                    preferred_element_type=jnp.float32)
        brows = []
        for b in range(B):
            heads = []
            for h in range(HQ_LOCAL):
                q = Q[b * SQ:(b + 1) * SQ,
                      h * DH:(h + 1) * DH].astype(jnp.bfloat16)
                k = k_ref[b, :, h, :]
                v = v_ref[b, :, h, :]
                s = lax.dot_general(
                    q, k, (((1,), (1,)), ((), ())),
                    preferred_element_type=jnp.float32) * 0.125
                m = jnp.max(s, axis=1, keepdims=True)
                w = jnp.exp(s - m)
                w = (w / jnp.sum(w, axis=1, keepdims=True)).astype(
                    jnp.bfloat16)
                heads.append(jnp.dot(w, v,
                                     preferred_element_type=jnp.float32))
            brows.append(jnp.concatenate(heads, axis=1))
        ctx = jnp.concatenate(brows, axis=0).astype(jnp.bfloat16)
        part = jnp.dot(ctx, wo_ref[...],
                       preferred_element_type=jnp.float32)
        pbf_ref[...] = part.astype(jnp.bfloat16).reshape(
            N_DEV, CHUNK, D_MODEL)

        rs_ref[pl.ds(me, 1)] = pbf_ref[pl.ds(me, 1)]

        pl.semaphore_wait(barrier, N_DEV - 1)

        sends1 = []
        for t in range(1, N_DEV):
            j = lax.rem(me + t, N_DEV)
            r = pltpu.make_async_remote_copy(
                src_ref=pbf_ref.at[j],
                dst_ref=rs_ref.at[me],
                send_sem=ssem1.at[t - 1],
                recv_sem=rsem1.at[me],
                device_id=(j,),
                device_id_type=pl.DeviceIdType.MESH,
            )
            r.start()
            sends1.append(r)
        for t in range(1, N_DEV):
            j = lax.rem(me + t, N_DEV)
            rr = pltpu.make_async_remote_copy(
                src_ref=pbf_ref.at[0],
                dst_ref=rs_ref.at[j],
                send_sem=ssem1.at[t - 1],
                recv_sem=rsem1.at[j],
                device_id=(j,),
                device_id_type=pl.DeviceIdType.MESH,
            )
            rr.wait_recv()

        red = jnp.sum(rs_ref[...].astype(jnp.float32), axis=0)
        red_ref[...] = red.astype(jnp.bfloat16)

        sends2 = []
        for t in range(1, N_DEV):
            j = lax.rem(me + t, N_DEV)
            r = pltpu.make_async_remote_copy(
                src_ref=red_ref,
                dst_ref=ag_ref.at[me],
                send_sem=ssem2.at[t - 1],
                recv_sem=rsem2.at[me],
                device_id=(j,),
                device_id_type=pl.DeviceIdType.MESH,
            )
            r.start()
            sends2.append(r)
        for t in range(1, N_DEV):
            j = lax.rem(me + t, N_DEV)
            rr = pltpu.make_async_remote_copy(
                src_ref=red_ref,
                dst_ref=ag_ref.at[j],
                send_sem=ssem2.at[t - 1],
                recv_sem=rsem2.at[j],
                device_id=(j,),
                device_id_type=pl.DeviceIdType.MESH,
            )
            rr.wait_recv()

        out_ref[...] = ag_ref[...].astype(jnp.float32).reshape(
            ROWS, D_MODEL)
        out_ref[pl.ds(me * CHUNK, CHUNK)] = red

        for r in sends1:
            r.wait_send()
        for r in sends2:
            r.wait_send()

    out2 = pl.pallas_call(
        body,
        out_shape=jax.ShapeDtypeStruct((ROWS, D_MODEL), jnp.float32),
        in_specs=[pl.BlockSpec(memory_space=pltpu.VMEM)] * 5,
        out_specs=pl.BlockSpec(memory_space=pltpu.VMEM),
        scratch_shapes=[
            pltpu.VMEM((N_DEV, CHUNK, D_MODEL), jnp.bfloat16),
            pltpu.VMEM((N_DEV, CHUNK, D_MODEL), jnp.bfloat16),
            pltpu.VMEM((CHUNK, D_MODEL), jnp.bfloat16),
            pltpu.VMEM((N_DEV, CHUNK, D_MODEL), jnp.bfloat16),
            pltpu.SemaphoreType.DMA((N_DEV - 1,)),
            pltpu.SemaphoreType.DMA((N_DEV,)),
            pltpu.SemaphoreType.DMA((N_DEV - 1,)),
            pltpu.SemaphoreType.DMA((N_DEV,)),
        ],
        compiler_params=pltpu.CompilerParams(collective_id=0),
    )(x2, Wq_b, K_b, V_b, Wo_b)
    return out2.reshape(B, SQ, D_MODEL)


# device time: 24739 ns/iter; 1.2220x vs baseline; 1.0761x over previous
import jax
import jax.numpy as jnp
from jax import lax
from jax.experimental import pallas as pl
from jax.experimental.pallas import tpu as pltpu

N_DEV = 32
B, SQ, D_MODEL = 2, 128, 512
HQ_LOCAL, DH = 4, 64
HD_LOCAL = HQ_LOCAL * DH
ROWS = B * SQ
CHUNK = ROWS // N_DEV


def kernel(x, Wq, K_ext, V_ext, Wo):
    me_out = lax.axis_index("i")
    Wq_s = lax.dynamic_slice(Wq, (0, me_out * HD_LOCAL), (D_MODEL, HD_LOCAL))
    Wo_s = lax.dynamic_slice(Wo, (me_out * HD_LOCAL, 0), (HD_LOCAL, D_MODEL))
    x2 = x.reshape(ROWS, D_MODEL)

    def body(x_ref, wq_ref, k_ref, v_ref, wo_ref, out_ref,
             pbf_ref, rs_ref, red_ref, ag_ref, ssem1, rsem1, ssem2, rsem2):
        me = lax.axis_index("i")

        barrier = pltpu.get_barrier_semaphore()
        for t in range(1, N_DEV):
            j = lax.rem(me + t, N_DEV)
            pl.semaphore_signal(
                barrier, inc=1,
                device_id=(j,), device_id_type=pl.DeviceIdType.MESH,
            )

        Q = jnp.dot(x_ref[...].astype(jnp.bfloat16),
                    wq_ref[...].astype(jnp.bfloat16),
                    preferred_element_type=jnp.float32)
        brows = []
        for b in range(B):
            heads = []
            for h in range(HQ_LOCAL):
                q = Q[b * SQ:(b + 1) * SQ,
                      h * DH:(h + 1) * DH].astype(jnp.bfloat16)
                k = k_ref[b, :, h, :].astype(jnp.bfloat16)
                v = v_ref[b, :, h, :].astype(jnp.bfloat16)
                s = lax.dot_general(
                    q, k, (((1,), (1,)), ((), ())),
                    preferred_element_type=jnp.float32) * 0.125
                m = jnp.max(s, axis=1, keepdims=True)
                w = jnp.exp(s - m)
                w = (w / jnp.sum(w, axis=1, keepdims=True)).astype(
                    jnp.bfloat16)
                heads.append(jnp.dot(w, v,
                                     preferred_element_type=jnp.float32))
            brows.append(jnp.concatenate(heads, axis=1))
        ctx = jnp.concatenate(brows, axis=0).astype(jnp.bfloat16)
        part = jnp.dot(ctx, wo_ref[...].astype(jnp.bfloat16),
                       preferred_element_type=jnp.float32)
        pbf_ref[...] = part.astype(jnp.bfloat16).reshape(
            N_DEV, CHUNK, D_MODEL)

        rs_ref[pl.ds(me, 1)] = pbf_ref[pl.ds(me, 1)]

        pl.semaphore_wait(barrier, N_DEV - 1)

        sends1 = []
        for t in range(1, N_DEV):
            j = lax.rem(me + t, N_DEV)
            r = pltpu.make_async_remote_copy(
                src_ref=pbf_ref.at[j],
                dst_ref=rs_ref.at[me],
                send_sem=ssem1.at[t - 1],
                recv_sem=rsem1.at[me],
                device_id=(j,),
                device_id_type=pl.DeviceIdType.MESH,
            )
            r.start()
            sends1.append(r)
        for t in range(1, N_DEV):
            j = lax.rem(me + t, N_DEV)
            rr = pltpu.make_async_remote_copy(
                src_ref=pbf_ref.at[0],
                dst_ref=rs_ref.at[j],
                send_sem=ssem1.at[t - 1],
                recv_sem=rsem1.at[j],
                device_id=(j,),
                device_id_type=pl.DeviceIdType.MESH,
            )
            rr.wait_recv()

        red = jnp.sum(rs_ref[...].astype(jnp.float32), axis=0)
        red_ref[...] = red.astype(jnp.bfloat16)

        sends2 = []
        for t in range(1, N_DEV):
            j = lax.rem(me + t, N_DEV)
            r = pltpu.make_async_remote_copy(
                src_ref=red_ref,
                dst_ref=ag_ref.at[me],
                send_sem=ssem2.at[t - 1],
                recv_sem=rsem2.at[me],
                device_id=(j,),
                device_id_type=pl.DeviceIdType.MESH,
            )
            r.start()
            sends2.append(r)
        for t in range(1, N_DEV):
            j = lax.rem(me + t, N_DEV)
            rr = pltpu.make_async_remote_copy(
                src_ref=red_ref,
                dst_ref=ag_ref.at[j],
                send_sem=ssem2.at[t - 1],
                recv_sem=rsem2.at[j],
                device_id=(j,),
                device_id_type=pl.DeviceIdType.MESH,
            )
            rr.wait_recv()

        out_ref[...] = ag_ref[...].astype(jnp.float32).reshape(
            ROWS, D_MODEL)
        out_ref[pl.ds(me * CHUNK, CHUNK)] = red

        for r in sends1:
            r.wait_send()
        for r in sends2:
            r.wait_send()

    out2 = pl.pallas_call(
        body,
        out_shape=jax.ShapeDtypeStruct((ROWS, D_MODEL), jnp.float32),
        in_specs=[pl.BlockSpec(memory_space=pltpu.VMEM)] * 5,
        out_specs=pl.BlockSpec(memory_space=pltpu.VMEM),
        scratch_shapes=[
            pltpu.VMEM((N_DEV, CHUNK, D_MODEL), jnp.bfloat16),
            pltpu.VMEM((N_DEV, CHUNK, D_MODEL), jnp.bfloat16),
            pltpu.VMEM((CHUNK, D_MODEL), jnp.bfloat16),
            pltpu.VMEM((N_DEV, CHUNK, D_MODEL), jnp.bfloat16),
            pltpu.SemaphoreType.DMA((N_DEV - 1,)),
            pltpu.SemaphoreType.DMA((N_DEV,)),
            pltpu.SemaphoreType.DMA((N_DEV - 1,)),
            pltpu.SemaphoreType.DMA((N_DEV,)),
        ],
        compiler_params=pltpu.CompilerParams(collective_id=0),
    )(x2, Wq_s, K_ext, V_ext, Wo_s)
    return out2.reshape(B, SQ, D_MODEL)
